# baseline (speedup 1.0000x reference)
import functools

import jax
import jax.numpy as jnp
import numpy as np
from jax import lax
from jax.experimental import pallas as pl
from jax.experimental.pallas import tpu as pltpu
from jax.experimental.pallas import tpu_sc as plsc

_B = 64
_C = 3
_H = 128
_W = 128
_Q = 512
_K = 21

_S = 16
_DQ = (_C * _H * _W) // _S
_KD = 441
_KDP = 512

_NW = 32

_OFF_B = (_Q - _B) * _S
_OFF_KA = _OFF_B + _B * _S
_OFF_KB = _OFF_KA + (_Q - _B)

_CHUNKS_A = (_Q - _B) * _S // 16
_CHUNKS_B = _B * _S // 16
_A_PER_W = _CHUNKS_A // _NW
_B_PER_W = _CHUNKS_B // _NW
_KA_W = (_Q - _B) // 16
_QC_PER_W = (_B * _S) // _NW


def _build_idxbuf():
    idx = jax.random.permutation(jax.random.key(42), _Q).astype(jnp.int32)
    lanes_a = jnp.tile(jnp.arange(_S, dtype=jnp.int32), _Q - _B)
    lanes_b = jnp.tile(jnp.arange(_S, dtype=jnp.int32), _B)
    suba = jnp.repeat(idx[_B:], _S) * _S + lanes_a
    subb = jnp.repeat(idx[:_B], _S) * _S + lanes_b
    return jnp.concatenate([suba, subb, idx[_B:], idx[:_B]])

_mesh = plsc.VectorSubcoreMesh(core_axis_name="c", subcore_axis_name="s")


@functools.partial(
    pl.kernel,
    out_type=[
        jax.ShapeDtypeStruct((_Q * _S, _DQ), jnp.float32),
        jax.ShapeDtypeStruct((_Q * _S, _DQ), jnp.float32),
        jax.ShapeDtypeStruct((_Q, _KDP), jnp.float32),
        jax.ShapeDtypeStruct((_B * _S, _DQ), jnp.float32),
        jax.ShapeDtypeStruct((_B * _S, _DQ), jnp.float32),
        jax.ShapeDtypeStruct((_B, _KDP), jnp.float32),
    ],
    mesh=_mesh,
    scratch_types=[
        pltpu.VMEM((16,), jnp.int32),
        pltpu.VMEM((16, _DQ), jnp.float32),
        pltpu.VMEM((16, _KDP), jnp.float32),
        pltpu.SemaphoreType.DMA,
    ],
)
def _sc_permute(idxbuf, q2, k2, ker2, query2, keyimg2, lr2,
                newq2, newk2, newker2, deqq2, deqk2, deqker2,
                idx_v, buf, kbuf, sem):
    w = lax.axis_index("s") * 2 + lax.axis_index("c")

    def gather16(tbl, dst, idx_off, dst_off, stage):
        pltpu.sync_copy(idxbuf.at[pl.ds(idx_off, 16)], idx_v)
        pltpu.async_copy(tbl.at[idx_v], stage, sem).wait()
        pltpu.sync_copy(stage, dst.at[pl.ds(dst_off, 16)])

    qc = w * _QC_PER_W
    pltpu.sync_copy(query2.at[pl.ds(qc, _QC_PER_W)], newq2.at[pl.ds(qc, _QC_PER_W)])
    pltpu.sync_copy(keyimg2.at[pl.ds(qc, _QC_PER_W)], newk2.at[pl.ds(qc, _QC_PER_W)])
    pltpu.sync_copy(lr2.at[pl.ds(w * 2, 2)], newker2.at[pl.ds(w * 2, 2)])

    for i in range(_A_PER_W):
        c = w * _A_PER_W + i
        gather16(q2, newq2, c * 16, _B * _S + c * 16, buf)
        gather16(k2, newk2, c * 16, _B * _S + c * 16, buf)
    for i in range(_B_PER_W):
        c = w * _B_PER_W + i
        gather16(q2, deqq2, _OFF_B + c * 16, c * 16, buf)
        gather16(k2, deqk2, _OFF_B + c * 16, c * 16, buf)

    @pl.when(w < _KA_W)
    def _():
        gather16(ker2, newker2, _OFF_KA + w * 16, _B + w * 16, kbuf)

    @pl.when(w >= _KA_W)
    def _():
        gather16(ker2, deqker2, _OFF_KB + (w - _KA_W) * 16, (w - _KA_W) * 16, kbuf)


def kernel(query, key_img, lr_gt_kernel, queue_q, queue_k, queue_ker):
    q2 = queue_q.reshape(_Q * _S, _DQ)
    k2 = queue_k.reshape(_Q * _S, _DQ)
    ker2 = jnp.pad(queue_ker.reshape(_Q, _KD), ((0, 0), (0, _KDP - _KD)))
    query2 = query.reshape(_B * _S, _DQ)
    keyimg2 = key_img.reshape(_B * _S, _DQ)
    lr2 = jnp.pad(lr_gt_kernel.reshape(_B, _KD), ((0, 0), (0, _KDP - _KD)))
    idxbuf = _build_idxbuf()

    newq2, newk2, newker2, deqq2, deqk2, deqker2 = _sc_permute(
        idxbuf, q2, k2, ker2, query2, keyimg2, lr2)

    new_qq = newq2.reshape(_Q, _C, _H, _W)
    new_qk = newk2.reshape(_Q, _C, _H, _W)
    new_qker = newker2[:, :_KD].reshape(_Q, 1, _K, _K)
    q_deq = deqq2.reshape(_B, _C, _H, _W)
    k_deq = deqk2.reshape(_B, _C, _H, _W)
    ker_deq = deqker2[:, :_KD].reshape(_B, 1, _K, _K)
    return (q_deq, k_deq, ker_deq, new_qq, new_qk, new_qker)


# speedup vs baseline: 1.1356x; 1.1356x over previous
import functools

import jax
import jax.numpy as jnp
import numpy as np
from jax import lax
from jax.experimental import pallas as pl
from jax.experimental.pallas import tpu as pltpu
from jax.experimental.pallas import tpu_sc as plsc

_B = 64
_C = 3
_H = 128
_W = 128
_Q = 512
_K = 21

_S = 16
_DQ = (_C * _H * _W) // _S
_KD = 441
_KDP = 512

_NW = 32

_OFF_B = (_Q - _B) * _S
_OFF_KA = _OFF_B + _B * _S
_OFF_KB = _OFF_KA + (_Q - _B)

_CH = 8
_A_PER_W = (_Q - _B) * _S // _CH // _NW
_B_PER_W = _B * _S // _CH // _NW
_KA_W = (_Q - _B) // 16
_QC_PER_W = (_B * _S) // _NW
_NB = 4


def _build_idxbuf():
    idx = jax.random.permutation(jax.random.key(42), _Q).astype(jnp.int32)
    lanes_a = jnp.tile(jnp.arange(_S, dtype=jnp.int32), _Q - _B)
    lanes_b = jnp.tile(jnp.arange(_S, dtype=jnp.int32), _B)
    suba = jnp.repeat(idx[_B:], _S) * _S + lanes_a
    subb = jnp.repeat(idx[:_B], _S) * _S + lanes_b
    return jnp.concatenate([suba, subb, idx[_B:], idx[:_B]])

_mesh = plsc.VectorSubcoreMesh(core_axis_name="c", subcore_axis_name="s")


@functools.partial(
    pl.kernel,
    out_type=[
        jax.ShapeDtypeStruct((_Q * _S, _DQ), jnp.float32),
        jax.ShapeDtypeStruct((_Q * _S, _DQ), jnp.float32),
        jax.ShapeDtypeStruct((_Q, _KDP), jnp.float32),
        jax.ShapeDtypeStruct((_B * _S, _DQ), jnp.float32),
        jax.ShapeDtypeStruct((_B * _S, _DQ), jnp.float32),
        jax.ShapeDtypeStruct((_B, _KDP), jnp.float32),
    ],
    mesh=_mesh,
    scratch_types=[
        pltpu.VMEM((256,), jnp.int32),
        pltpu.VMEM((16,), jnp.int32),
        pltpu.VMEM((_NB, _CH, _DQ), jnp.float32),
        pltpu.VMEM((16, _KDP), jnp.float32),
        pltpu.SemaphoreType.DMA,
        pltpu.SemaphoreType.DMA,
        pltpu.SemaphoreType.DMA,
    ],
)
def _sc_permute(idxbuf, q2, k2, ker2, query2, keyimg2, lr2,
                newq2, newk2, newker2, deqq2, deqk2, deqker2,
                idx_v, kidx_v, bufs, kbuf, gsem, wsem, hsem):
    w = lax.axis_index("s") * 2 + lax.axis_index("c")

    qc = w * _QC_PER_W
    h0 = pltpu.async_copy(query2.at[pl.ds(qc, _QC_PER_W)],
                          newq2.at[pl.ds(qc, _QC_PER_W)], hsem)
    h1 = pltpu.async_copy(keyimg2.at[pl.ds(qc, _QC_PER_W)],
                          newk2.at[pl.ds(qc, _QC_PER_W)], hsem)
    h2 = pltpu.async_copy(lr2.at[pl.ds(w * 2, 2)],
                          newker2.at[pl.ds(w * 2, 2)], hsem)

    pltpu.sync_copy(idxbuf.at[pl.ds(w * (_A_PER_W * _CH), _A_PER_W * _CH)],
                    idx_v.at[pl.ds(0, _A_PER_W * _CH)])
    pltpu.sync_copy(idxbuf.at[pl.ds(_OFF_B + w * (_B_PER_W * _CH), _B_PER_W * _CH)],
                    idx_v.at[pl.ds(_A_PER_W * _CH, _B_PER_W * _CH)])

    tasks = []
    for i in range(_A_PER_W):
        c = w * _A_PER_W + i
        tasks.append((q2, newq2, i * _CH, _B * _S + c * _CH))
        tasks.append((k2, newk2, i * _CH, _B * _S + c * _CH))
    for i in range(_B_PER_W):
        c = w * _B_PER_W + i
        tasks.append((q2, deqq2, _A_PER_W * _CH + i * _CH, c * _CH))
        tasks.append((k2, deqk2, _A_PER_W * _CH + i * _CH, c * _CH))

    n = len(tasks)
    hg = [None] * n
    hw = [None] * n

    def fire_gather(t):
        tbl, _, ioff, _ = tasks[t]
        hg[t] = pltpu.async_copy(tbl.at[idx_v.at[pl.ds(ioff, _CH)]],
                                 bufs.at[t % _NB], gsem)

    def fire_write(t):
        _, dst, _, doff = tasks[t]
        hg[t].wait()
        hw[t] = pltpu.async_copy(bufs.at[t % _NB],
                                 dst.at[pl.ds(doff, _CH)], wsem)

    for t in range(n + 2):
        if t < n:
            if t >= _NB:
                hw[t - _NB].wait()
            fire_gather(t)
        if t >= 2 and t - 2 < n:
            fire_write(t - 2)
    for t in range(n - _NB, n):
        hw[t].wait()

    def ker_chunk(idx_off, dst, dst_off):
        pltpu.sync_copy(idxbuf.at[pl.ds(idx_off, 16)], kidx_v)
        pltpu.async_copy(ker2.at[kidx_v], kbuf, gsem).wait()
        pltpu.sync_copy(kbuf, dst.at[pl.ds(dst_off, 16)])

    @pl.when(w < _KA_W)
    def _():
        ker_chunk(_OFF_KA + w * 16, newker2, _B + w * 16)

    @pl.when(w >= _KA_W)
    def _():
        ker_chunk(_OFF_KB + (w - _KA_W) * 16, deqker2, (w - _KA_W) * 16)

    h0.wait()
    h1.wait()
    h2.wait()


def kernel(query, key_img, lr_gt_kernel, queue_q, queue_k, queue_ker):
    q2 = queue_q.reshape(_Q * _S, _DQ)
    k2 = queue_k.reshape(_Q * _S, _DQ)
    ker2 = jnp.pad(queue_ker.reshape(_Q, _KD), ((0, 0), (0, _KDP - _KD)))
    query2 = query.reshape(_B * _S, _DQ)
    keyimg2 = key_img.reshape(_B * _S, _DQ)
    lr2 = jnp.pad(lr_gt_kernel.reshape(_B, _KD), ((0, 0), (0, _KDP - _KD)))
    idxbuf = _build_idxbuf()

    newq2, newk2, newker2, deqq2, deqk2, deqker2 = _sc_permute(
        idxbuf, q2, k2, ker2, query2, keyimg2, lr2)

    new_qq = newq2.reshape(_Q, _C, _H, _W)
    new_qk = newk2.reshape(_Q, _C, _H, _W)
    new_qker = newker2[:, :_KD].reshape(_Q, 1, _K, _K)
    q_deq = deqq2.reshape(_B, _C, _H, _W)
    k_deq = deqk2.reshape(_B, _C, _H, _W)
    ker_deq = deqker2[:, :_KD].reshape(_B, 1, _K, _K)
    return (q_deq, k_deq, ker_deq, new_qq, new_qk, new_qker)
